# baseline (speedup 1.0000x reference)
import functools

import jax
import jax.numpy as jnp
from jax import lax
from jax.experimental import pallas as pl
from jax.experimental.pallas import tpu as pltpu
from jax.experimental.pallas import tpu_sc as plsc

NUM_CLASSES = 19
GAMMA = 1.0
EPS = 1e-07
IGNORE = 255

LN2 = 0.6931471805599453
SQRT2 = 1.4142135623730951

NC = 2
NS = 16
NW = NC * NS
L = 16

BATCH = 4
H = 512
W = 512

H_SC = 0
H_TC = H - H_SC

NBAND = 8
NSTRIPE = 4
RB = 24
SW = W // NSTRIPE
RPC = 8
NCHUNK = 3
CHUNK_VECS = RPC * (SW // L)

HB = 128
TC_GRID_H = H_TC // HB


def _log_f32(p):
    bits = lax.bitcast_convert_type(p, jnp.int32)
    e = ((bits >> 23) & 0xFF) - 127
    mbits = (bits & 0x7FFFFF) | (127 << 23)
    m = lax.bitcast_convert_type(mbits, jnp.float32)
    big = m > SQRT2
    m = jnp.where(big, m * 0.5, m)
    e = e + jnp.where(big, jnp.ones_like(e), jnp.zeros_like(e))
    ef = e.astype(jnp.float32)
    u = (m - 1.0) / (m + 1.0)
    u2 = u * u
    poly = 2.0 * u * (1.0 + u2 * (1.0 / 3.0 + u2 * (1.0 / 5.0 + u2 * (1.0 / 7.0))))
    return ef * LN2 + poly


def _pixel_vec(xbuf, tbuf, r, col, lane_iota):
    xs = [xbuf[cls, r, pl.ds(col, L)] for cls in range(NUM_CLASSES)]
    t = tbuf[r, pl.ds(col, L)]
    m = xs[0]
    for cls in range(1, NUM_CLASSES):
        m = jnp.maximum(m, xs[cls])
    z = jnp.exp(xs[0] - m)
    for cls in range(1, NUM_CLASSES):
        z = z + jnp.exp(xs[cls] - m)
    tg = jnp.minimum(t, NUM_CLASSES - 1)
    r_vec = jnp.zeros((L,), jnp.int32) + r
    s = plsc.load_gather(xbuf, [tg, r_vec, col + lane_iota])
    p = jnp.exp(s - m) / z
    p = jnp.minimum(jnp.maximum(p, EPS), 1.0 - EPS)
    log_p = _log_f32(p)
    valid = t != IGNORE
    zero = jnp.zeros((L,), jnp.float32)
    one = jnp.ones((L,), jnp.float32)
    contrib = jnp.where(valid, (1.0 - p) * log_p, zero)
    return contrib, jnp.where(valid, one, zero)


def _chunk_loop(xbuf, tbuf, carry):
    lane_iota = lax.iota(jnp.int32, L)
    cpr = SW // L

    def it(i, c):
        al, ac = c
        for u in range(2):
            v = i * 2 + u
            r = v // cpr
            col = (v % cpr) * L
            contrib, cnt = _pixel_vec(xbuf, tbuf, r, col, lane_iota)
            al = al - contrib
            ac = ac + cnt
        return al, ac

    return lax.fori_loop(0, CHUNK_VECS // 2, it, carry)


def _sc_body(x_hbm, t_hbm, out_hbm, xbuf0, xbuf1, tbuf0, tbuf1, accbuf,
             xsem0, xsem1, tsem0, tsem1):
    cid = lax.axis_index("c")
    sid = lax.axis_index("s")
    wid = sid * NC + cid
    accbuf[0, pl.ds(0, L)] = jnp.zeros((L,), jnp.float32)
    accbuf[1, pl.ds(0, L)] = jnp.zeros((L,), jnp.float32)
    pltpu.sync_copy(accbuf, out_hbm.at[wid])


def _sc_loss(x, t):
    mesh = plsc.VectorSubcoreMesh(core_axis_name="c", subcore_axis_name="s")
    run = functools.partial(
        pl.kernel,
        out_type=jax.ShapeDtypeStruct((NW, 2, L), jnp.float32),
        mesh=mesh,
        compiler_params=pltpu.CompilerParams(needs_layout_passes=False, skip_device_barrier=True),
        scratch_types=[
            pltpu.VMEM((NUM_CLASSES, RPC, SW), jnp.float32),
            pltpu.VMEM((NUM_CLASSES, RPC, SW), jnp.float32),
            pltpu.VMEM((RPC, SW), jnp.int32),
            pltpu.VMEM((RPC, SW), jnp.int32),
            pltpu.VMEM((2, L), jnp.float32),
            pltpu.SemaphoreType.DMA,
            pltpu.SemaphoreType.DMA,
            pltpu.SemaphoreType.DMA,
            pltpu.SemaphoreType.DMA,
        ],
    )(_sc_body)
    return run(x, t)


def _tc_body(x_ref, t_ref, sum_ref, cnt_ref):
    b = pl.program_id(0)
    h = pl.program_id(1)

    @pl.when(jnp.logical_and(b == 0, h == 0))
    def _():
        sum_ref[...] = jnp.zeros_like(sum_ref)
        cnt_ref[...] = jnp.zeros_like(cnt_ref)

    t = t_ref[0]
    m = x_ref[0, 0]
    for c in range(1, NUM_CLASSES):
        m = jnp.maximum(m, x_ref[0, c])
    z = jnp.zeros((HB, W), jnp.float32)
    s = jnp.zeros((HB, W), jnp.float32)
    for c in range(NUM_CLASSES):
        xc = x_ref[0, c]
        z = z + jnp.exp(xc - m)
        s = s + jnp.where(t == c, xc, 0.0)
    p = jnp.exp(s - m) / z
    p = jnp.clip(p, EPS, 1.0 - EPS)
    log_p = jnp.log(p)
    valid = t != IGNORE
    loss = jnp.where(valid, -(1.0 - p) * log_p, 0.0)
    cnt = jnp.where(valid, 1.0, 0.0)
    sum_ref[...] += jnp.sum(loss.reshape(-1, 8, 128), axis=0)
    cnt_ref[...] += jnp.sum(cnt.reshape(-1, 8, 128), axis=0)


def _tc_loss(x, t):
    return pl.pallas_call(
        _tc_body,
        grid=(BATCH, TC_GRID_H),
        in_specs=[
            pl.BlockSpec((1, NUM_CLASSES, HB, W), lambda b, h: (b, 0, h, 0)),
            pl.BlockSpec((1, HB, W), lambda b, h: (b, h, 0)),
        ],
        out_specs=[
            pl.BlockSpec((8, 128), lambda b, h: (0, 0)),
            pl.BlockSpec((8, 128), lambda b, h: (0, 0)),
        ],
        out_shape=[
            jax.ShapeDtypeStruct((8, 128), jnp.float32),
            jax.ShapeDtypeStruct((8, 128), jnp.float32),
        ],
    )(x, t)


@jax.jit
def _loss(input, target):
    sc_parts = _sc_loss(input, target)
    tc_sum, tc_cnt = _tc_loss(input, target)
    total = jnp.sum(sc_parts[:, 0, :]) + jnp.sum(tc_sum)
    count = jnp.sum(sc_parts[:, 1, :]) + jnp.sum(tc_cnt)
    return total / jnp.maximum(count, 1.0)


def kernel(input, target):
    return _loss(input, target)


# speedup vs baseline: 1.1126x; 1.1126x over previous
import functools

import jax
import jax.numpy as jnp
from jax import lax
from jax.experimental import pallas as pl
from jax.experimental.pallas import tpu as pltpu
from jax.experimental.pallas import tpu_sc as plsc

NUM_CLASSES = 19
GAMMA = 1.0
EPS = 1e-07
IGNORE = 255

LN2 = 0.6931471805599453
SQRT2 = 1.4142135623730951

NC = 2
NS = 16
NW = NC * NS
L = 16

BATCH = 4
H = 512
W = 512

H_SC = 64
H_TC = H - H_SC

NBAND = 8
NSTRIPE = 4
RB = H_SC // NBAND
SW = W // NSTRIPE
RPC = 8
NCHUNK = RB // RPC
CHUNK_VECS = RPC * (SW // L)

HB = 112
TC_GRID_H = H_TC // HB


def _log_f32(p):
    bits = lax.bitcast_convert_type(p, jnp.int32)
    e = ((bits >> 23) & 0xFF) - 127
    mbits = (bits & 0x7FFFFF) | (127 << 23)
    m = lax.bitcast_convert_type(mbits, jnp.float32)
    big = m > SQRT2
    m = jnp.where(big, m * 0.5, m)
    e = e + jnp.where(big, jnp.ones_like(e), jnp.zeros_like(e))
    ef = e.astype(jnp.float32)
    u = (m - 1.0) / (m + 1.0)
    u2 = u * u
    poly = 2.0 * u * (1.0 + u2 * (1.0 / 3.0 + u2 * (1.0 / 5.0 + u2 * (1.0 / 7.0))))
    return ef * LN2 + poly


def _pixel_vec(xbuf, tbuf, r, col, lane_iota):
    xs = [xbuf[cls, r, pl.ds(col, L)] for cls in range(NUM_CLASSES)]
    t = tbuf[r, pl.ds(col, L)]
    m = xs[0]
    for cls in range(1, NUM_CLASSES):
        m = jnp.maximum(m, xs[cls])
    z = jnp.exp(xs[0] - m)
    for cls in range(1, NUM_CLASSES):
        z = z + jnp.exp(xs[cls] - m)
    tg = jnp.minimum(t, NUM_CLASSES - 1)
    r_vec = jnp.zeros((L,), jnp.int32) + r
    s = plsc.load_gather(xbuf, [tg, r_vec, col + lane_iota])
    p = jnp.exp(s - m) / z
    p = jnp.minimum(jnp.maximum(p, EPS), 1.0 - EPS)
    log_p = _log_f32(p)
    valid = t != IGNORE
    zero = jnp.zeros((L,), jnp.float32)
    one = jnp.ones((L,), jnp.float32)
    contrib = jnp.where(valid, (1.0 - p) * log_p, zero)
    return contrib, jnp.where(valid, one, zero)


def _chunk_loop(xbuf, tbuf, carry):
    lane_iota = lax.iota(jnp.int32, L)
    cpr = SW // L

    def it(i, c):
        al, ac = c
        for u in range(2):
            v = i * 2 + u
            r = v // cpr
            col = (v % cpr) * L
            contrib, cnt = _pixel_vec(xbuf, tbuf, r, col, lane_iota)
            al = al - contrib
            ac = ac + cnt
        return al, ac

    return lax.fori_loop(0, CHUNK_VECS // 2, it, carry)


def _sc_body(x_hbm, t_hbm, out_hbm, xbuf0, xbuf1, tbuf0, tbuf1, accbuf,
             xsem0, xsem1, tsem0, tsem1):
    cid = lax.axis_index("c")
    sid = lax.axis_index("s")
    wid = sid * NC + cid
    band = wid // NSTRIPE
    stripe = wid % NSTRIPE
    row_base = H_TC + band * RB
    col0 = pl.multiple_of(stripe * SW, SW)

    bufs = ((xbuf0, tbuf0, xsem0, tsem0), (xbuf1, tbuf1, xsem1, tsem1))
    nsteps = BATCH * NCHUNK

    def issue(step, bufset):
        s = jnp.minimum(step, nsteps - 1)
        b = s // NCHUNK
        j = s - b * NCHUNK
        row0 = pl.multiple_of(row_base + j * RPC, 8)
        cx = pltpu.async_copy(
            x_hbm.at[b, :, pl.ds(row0, RPC), pl.ds(col0, SW)],
            bufset[0], bufset[2])
        ct = pltpu.async_copy(
            t_hbm.at[b, pl.ds(row0, RPC), pl.ds(col0, SW)],
            bufset[1], bufset[3])
        return cx, ct

    issue(0, bufs[0])
    issue(1, bufs[1])

    def ring(g, carry):
        acc = carry
        for par in range(2):
            bufset = bufs[par]
            pltpu.make_async_copy(
                x_hbm.at[0, :, pl.ds(0, RPC), pl.ds(0, SW)],
                bufset[0], bufset[2]).wait()
            pltpu.make_async_copy(
                t_hbm.at[0, pl.ds(0, RPC), pl.ds(0, SW)],
                bufset[1], bufset[3]).wait()
            acc = _chunk_loop(bufset[0], bufset[1], acc)
            issue(2 * g + par + 2, bufset)
        return acc

    acc = lax.fori_loop(
        0, nsteps // 2, ring,
        (jnp.zeros((L,), jnp.float32), jnp.zeros((L,), jnp.float32)))

    for bufset in bufs:
        pltpu.make_async_copy(
            x_hbm.at[0, :, pl.ds(0, RPC), pl.ds(0, SW)],
            bufset[0], bufset[2]).wait()
        pltpu.make_async_copy(
            t_hbm.at[0, pl.ds(0, RPC), pl.ds(0, SW)],
            bufset[1], bufset[3]).wait()

    accbuf[0, pl.ds(0, L)] = acc[0]
    accbuf[1, pl.ds(0, L)] = acc[1]
    pltpu.sync_copy(accbuf, out_hbm.at[wid])


def _sc_loss(x, t):
    mesh = plsc.VectorSubcoreMesh(core_axis_name="c", subcore_axis_name="s")
    run = functools.partial(
        pl.kernel,
        out_type=jax.ShapeDtypeStruct((NW, 2, L), jnp.float32),
        mesh=mesh,
        compiler_params=pltpu.CompilerParams(needs_layout_passes=False, skip_device_barrier=True),
        scratch_types=[
            pltpu.VMEM((NUM_CLASSES, RPC, SW), jnp.float32),
            pltpu.VMEM((NUM_CLASSES, RPC, SW), jnp.float32),
            pltpu.VMEM((RPC, SW), jnp.int32),
            pltpu.VMEM((RPC, SW), jnp.int32),
            pltpu.VMEM((2, L), jnp.float32),
            pltpu.SemaphoreType.DMA,
            pltpu.SemaphoreType.DMA,
            pltpu.SemaphoreType.DMA,
            pltpu.SemaphoreType.DMA,
        ],
    )(_sc_body)
    return run(x, t)


def _tc_body(x_ref, t_ref, sum_ref, cnt_ref):
    b = pl.program_id(0)
    h = pl.program_id(1)

    acc_l = jnp.zeros((8, W), jnp.float32)
    acc_c = jnp.zeros((8, W), jnp.float32)
    for h8 in range(HB // 8):
        r0 = h8 * 8
        t8 = t_ref[0, pl.ds(r0, 8), :]
        m = x_ref[0, 0, pl.ds(r0, 8), :]
        for c in range(1, NUM_CLASSES):
            m = jnp.maximum(m, x_ref[0, c, pl.ds(r0, 8), :])
        z = jnp.zeros((8, W), jnp.float32)
        s = jnp.zeros((8, W), jnp.float32)
        for c in range(NUM_CLASSES):
            xc = x_ref[0, c, pl.ds(r0, 8), :]
            z = z + jnp.exp(xc - m)
            s = s + jnp.where(t8 == c, xc, 0.0)
        p = jnp.exp(s - m) / z
        p = jnp.clip(p, EPS, 1.0 - EPS)
        log_p = jnp.log(p)
        valid = t8 != IGNORE
        acc_l = acc_l + jnp.where(valid, -(1.0 - p) * log_p, 0.0)
        acc_c = acc_c + jnp.where(valid, 1.0, 0.0)

    @pl.when(jnp.logical_and(b == 0, h == 0))
    def _():
        sum_ref[...] = jnp.zeros_like(sum_ref)
        cnt_ref[...] = jnp.zeros_like(cnt_ref)

    sum_ref[...] += acc_l
    cnt_ref[...] += acc_c


def _tc_loss(x, t):
    return pl.pallas_call(
        _tc_body,
        grid=(BATCH, TC_GRID_H),
        in_specs=[
            pl.BlockSpec((1, NUM_CLASSES, HB, W), lambda b, h: (b, 0, h, 0)),
            pl.BlockSpec((1, HB, W), lambda b, h: (b, h, 0)),
        ],
        out_specs=[
            pl.BlockSpec((8, W), lambda b, h: (0, 0)),
            pl.BlockSpec((8, W), lambda b, h: (0, 0)),
        ],
        out_shape=[
            jax.ShapeDtypeStruct((8, W), jnp.float32),
            jax.ShapeDtypeStruct((8, W), jnp.float32),
        ],
    )(x, t)


@jax.jit
def _loss(input, target):
    sc_parts = _sc_loss(input, target)
    tc_sum, tc_cnt = _tc_loss(input, target)
    total = jnp.sum(sc_parts[:, 0, :]) + jnp.sum(tc_sum)
    count = jnp.sum(sc_parts[:, 1, :]) + jnp.sum(tc_cnt)
    return total / jnp.maximum(count, 1.0)


def kernel(input, target):
    return _loss(input, target)


# speedup vs baseline: 1.1525x; 1.0358x over previous
import functools

import jax
import jax.numpy as jnp
from jax import lax
from jax.experimental import pallas as pl
from jax.experimental.pallas import tpu as pltpu
from jax.experimental.pallas import tpu_sc as plsc

NUM_CLASSES = 19
GAMMA = 1.0
EPS = 1e-07
IGNORE = 255

LN2 = 0.6931471805599453
SQRT2 = 1.4142135623730951

NC = 2
NS = 16
NW = NC * NS
L = 16

BATCH = 4
H = 512
W = 512

H_SC = 128
H_TC = H - H_SC

NBAND = 8
NSTRIPE = 4
RB = H_SC // NBAND
SW = W // NSTRIPE
RPC = 8
NCHUNK = RB // RPC
CHUNK_VECS = RPC * (SW // L)

HB = 128
TC_GRID_H = H_TC // HB


def _log_f32(p):
    bits = lax.bitcast_convert_type(p, jnp.int32)
    e = ((bits >> 23) & 0xFF) - 127
    mbits = (bits & 0x7FFFFF) | (127 << 23)
    m = lax.bitcast_convert_type(mbits, jnp.float32)
    big = m > SQRT2
    m = jnp.where(big, m * 0.5, m)
    e = e + jnp.where(big, jnp.ones_like(e), jnp.zeros_like(e))
    ef = e.astype(jnp.float32)
    u = (m - 1.0) / (m + 1.0)
    u2 = u * u
    poly = 2.0 * u * (1.0 + u2 * (1.0 / 3.0 + u2 * (1.0 / 5.0 + u2 * (1.0 / 7.0))))
    return ef * LN2 + poly


def _pixel_vec(xbuf, tbuf, r, col, lane_iota):
    xs = [xbuf[cls, r, pl.ds(col, L)] for cls in range(NUM_CLASSES)]
    t = tbuf[r, pl.ds(col, L)]
    m = xs[0]
    for cls in range(1, NUM_CLASSES):
        m = jnp.maximum(m, xs[cls])
    z = jnp.exp(xs[0] - m)
    for cls in range(1, NUM_CLASSES):
        z = z + jnp.exp(xs[cls] - m)
    tg = jnp.minimum(t, NUM_CLASSES - 1)
    r_vec = jnp.zeros((L,), jnp.int32) + r
    s = plsc.load_gather(xbuf, [tg, r_vec, col + lane_iota])
    p = jnp.exp(s - m) / z
    p = jnp.minimum(jnp.maximum(p, EPS), 1.0 - EPS)
    log_p = _log_f32(p)
    valid = t != IGNORE
    zero = jnp.zeros((L,), jnp.float32)
    one = jnp.ones((L,), jnp.float32)
    contrib = jnp.where(valid, (1.0 - p) * log_p, zero)
    return contrib, jnp.where(valid, one, zero)


def _chunk_loop(xbuf, tbuf, carry):
    lane_iota = lax.iota(jnp.int32, L)
    cpr = SW // L

    def it(i, c):
        al, ac = c
        for u in range(2):
            v = i * 2 + u
            r = v // cpr
            col = (v % cpr) * L
            contrib, cnt = _pixel_vec(xbuf, tbuf, r, col, lane_iota)
            al = al - contrib
            ac = ac + cnt
        return al, ac

    return lax.fori_loop(0, CHUNK_VECS // 2, it, carry)


def _sc_body(x_hbm, t_hbm, out_hbm, xbuf0, xbuf1, tbuf0, tbuf1, accbuf,
             xsem0, xsem1, tsem0, tsem1):
    cid = lax.axis_index("c")
    sid = lax.axis_index("s")
    wid = sid * NC + cid
    band = wid // NSTRIPE
    stripe = wid % NSTRIPE
    row_base = H_TC + band * RB
    col0 = pl.multiple_of(stripe * SW, SW)

    bufs = ((xbuf0, tbuf0, xsem0, tsem0), (xbuf1, tbuf1, xsem1, tsem1))
    nsteps = BATCH * NCHUNK

    def issue(step, bufset):
        s = jnp.minimum(step, nsteps - 1)
        b = s // NCHUNK
        j = s - b * NCHUNK
        row0 = pl.multiple_of(row_base + j * RPC, 8)
        cx = pltpu.async_copy(
            x_hbm.at[b, :, pl.ds(row0, RPC), pl.ds(col0, SW)],
            bufset[0], bufset[2])
        ct = pltpu.async_copy(
            t_hbm.at[b, pl.ds(row0, RPC), pl.ds(col0, SW)],
            bufset[1], bufset[3])
        return cx, ct

    issue(0, bufs[0])
    issue(1, bufs[1])

    def ring(g, carry):
        acc = carry
        for par in range(2):
            bufset = bufs[par]
            pltpu.make_async_copy(
                x_hbm.at[0, :, pl.ds(0, RPC), pl.ds(0, SW)],
                bufset[0], bufset[2]).wait()
            pltpu.make_async_copy(
                t_hbm.at[0, pl.ds(0, RPC), pl.ds(0, SW)],
                bufset[1], bufset[3]).wait()
            acc = _chunk_loop(bufset[0], bufset[1], acc)
            issue(2 * g + par + 2, bufset)
        return acc

    acc = lax.fori_loop(
        0, nsteps // 2, ring,
        (jnp.zeros((L,), jnp.float32), jnp.zeros((L,), jnp.float32)))

    for bufset in bufs:
        pltpu.make_async_copy(
            x_hbm.at[0, :, pl.ds(0, RPC), pl.ds(0, SW)],
            bufset[0], bufset[2]).wait()
        pltpu.make_async_copy(
            t_hbm.at[0, pl.ds(0, RPC), pl.ds(0, SW)],
            bufset[1], bufset[3]).wait()

    accbuf[0, pl.ds(0, L)] = acc[0]
    accbuf[1, pl.ds(0, L)] = acc[1]
    pltpu.sync_copy(accbuf, out_hbm.at[wid])


def _sc_loss(x, t):
    mesh = plsc.VectorSubcoreMesh(core_axis_name="c", subcore_axis_name="s")
    run = functools.partial(
        pl.kernel,
        out_type=jax.ShapeDtypeStruct((NW, 2, L), jnp.float32),
        mesh=mesh,
        compiler_params=pltpu.CompilerParams(needs_layout_passes=False, skip_device_barrier=True),
        scratch_types=[
            pltpu.VMEM((NUM_CLASSES, RPC, SW), jnp.float32),
            pltpu.VMEM((NUM_CLASSES, RPC, SW), jnp.float32),
            pltpu.VMEM((RPC, SW), jnp.int32),
            pltpu.VMEM((RPC, SW), jnp.int32),
            pltpu.VMEM((2, L), jnp.float32),
            pltpu.SemaphoreType.DMA,
            pltpu.SemaphoreType.DMA,
            pltpu.SemaphoreType.DMA,
            pltpu.SemaphoreType.DMA,
        ],
    )(_sc_body)
    return run(x, t)


def _tc_body(x_ref, t_ref, sum_ref, cnt_ref):
    b = pl.program_id(0)
    h = pl.program_id(1)

    acc_l = jnp.zeros((8, W), jnp.float32)
    acc_c = jnp.zeros((8, W), jnp.float32)
    for h8 in range(HB // 8):
        r0 = h8 * 8
        t8 = t_ref[0, pl.ds(r0, 8), :]
        m = x_ref[0, 0, pl.ds(r0, 8), :]
        for c in range(1, NUM_CLASSES):
            m = jnp.maximum(m, x_ref[0, c, pl.ds(r0, 8), :])
        z = jnp.zeros((8, W), jnp.float32)
        s = jnp.zeros((8, W), jnp.float32)
        for c in range(NUM_CLASSES):
            xc = x_ref[0, c, pl.ds(r0, 8), :]
            z = z + jnp.exp(xc - m)
            s = s + jnp.where(t8 == c, xc, 0.0)
        p = jnp.exp(s - m) / z
        p = jnp.clip(p, EPS, 1.0 - EPS)
        log_p = jnp.log(p)
        valid = t8 != IGNORE
        acc_l = acc_l + jnp.where(valid, -(1.0 - p) * log_p, 0.0)
        acc_c = acc_c + jnp.where(valid, 1.0, 0.0)

    @pl.when(jnp.logical_and(b == 0, h == 0))
    def _():
        sum_ref[...] = jnp.zeros_like(sum_ref)
        cnt_ref[...] = jnp.zeros_like(cnt_ref)

    sum_ref[...] += acc_l
    cnt_ref[...] += acc_c


def _tc_loss(x, t):
    return pl.pallas_call(
        _tc_body,
        grid=(BATCH, TC_GRID_H),
        in_specs=[
            pl.BlockSpec((1, NUM_CLASSES, HB, W), lambda b, h: (b, 0, h, 0)),
            pl.BlockSpec((1, HB, W), lambda b, h: (b, h, 0)),
        ],
        out_specs=[
            pl.BlockSpec((8, W), lambda b, h: (0, 0)),
            pl.BlockSpec((8, W), lambda b, h: (0, 0)),
        ],
        out_shape=[
            jax.ShapeDtypeStruct((8, W), jnp.float32),
            jax.ShapeDtypeStruct((8, W), jnp.float32),
        ],
    )(x, t)


@jax.jit
def _loss(input, target):
    sc_parts = _sc_loss(input, target)
    tc_sum, tc_cnt = _tc_loss(input, target)
    total = jnp.sum(sc_parts[:, 0, :]) + jnp.sum(tc_sum)
    count = jnp.sum(sc_parts[:, 1, :]) + jnp.sum(tc_cnt)
    return total / jnp.maximum(count, 1.0)


def kernel(input, target):
    return _loss(input, target)


# speedup vs baseline: 1.1754x; 1.0199x over previous
import functools

import jax
import jax.numpy as jnp
from jax import lax
from jax.experimental import pallas as pl
from jax.experimental.pallas import tpu as pltpu
from jax.experimental.pallas import tpu_sc as plsc

NUM_CLASSES = 19
GAMMA = 1.0
EPS = 1e-07
IGNORE = 255

LN2 = 0.6931471805599453
SQRT2 = 1.4142135623730951

NC = 2
NS = 16
NW = NC * NS
L = 16

BATCH = 4
H = 512
W = 512

H_SC = 128
H_TC = H - H_SC

NBAND = 8
NSTRIPE = 4
RB = H_SC // NBAND
SW = W // NSTRIPE
RPC = 8
NCHUNK = RB // RPC
CHUNK_VECS = RPC * (SW // L)

HB = 128
TC_GRID_H = H_TC // HB


def _log_f32(p):
    bits = lax.bitcast_convert_type(p, jnp.int32)
    e = ((bits >> 23) & 0xFF) - 127
    mbits = (bits & 0x7FFFFF) | (127 << 23)
    m = lax.bitcast_convert_type(mbits, jnp.float32)
    big = m > SQRT2
    m = jnp.where(big, m * 0.5, m)
    e = e + jnp.where(big, jnp.ones_like(e), jnp.zeros_like(e))
    ef = e.astype(jnp.float32)
    u = (m - 1.0) / (m + 1.0)
    u2 = u * u
    poly = 2.0 * u * (1.0 + u2 * (1.0 / 3.0 + u2 * (1.0 / 5.0 + u2 * (1.0 / 7.0))))
    return ef * LN2 + poly


def _pixel_vec(xbuf, tbuf, r, col, lane_iota):
    xs = [xbuf[cls, r, pl.ds(col, L)] for cls in range(NUM_CLASSES)]
    t = tbuf[r, pl.ds(col, L)]
    m = xs[0]
    for cls in range(1, NUM_CLASSES):
        m = jnp.maximum(m, xs[cls])
    z = jnp.exp(xs[0] - m)
    for cls in range(1, NUM_CLASSES):
        z = z + jnp.exp(xs[cls] - m)
    tg = jnp.minimum(t, NUM_CLASSES - 1)
    r_vec = jnp.zeros((L,), jnp.int32) + r
    s = plsc.load_gather(xbuf, [tg, r_vec, col + lane_iota])
    p = jnp.exp(s - m) / z
    p = jnp.minimum(jnp.maximum(p, EPS), 1.0 - EPS)
    log_p = _log_f32(p)
    valid = t != IGNORE
    zero = jnp.zeros((L,), jnp.float32)
    one = jnp.ones((L,), jnp.float32)
    contrib = jnp.where(valid, (1.0 - p) * log_p, zero)
    return contrib, jnp.where(valid, one, zero)


def _chunk_loop(xbuf, tbuf, carry):
    lane_iota = lax.iota(jnp.int32, L)
    cpr = SW // L

    def it(i, c):
        al, ac = c
        for u in range(2):
            v = i * 2 + u
            r = v // cpr
            col = (v % cpr) * L
            contrib, cnt = _pixel_vec(xbuf, tbuf, r, col, lane_iota)
            al = al - contrib
            ac = ac + cnt
        return al, ac

    return lax.fori_loop(0, CHUNK_VECS // 2, it, carry)


def _sc_body(x_hbm, t_hbm, out_hbm, xbuf0, xbuf1, tbuf0, tbuf1, accbuf,
             xsem0, xsem1, tsem0, tsem1):
    cid = lax.axis_index("c")
    sid = lax.axis_index("s")
    wid = sid * NC + cid
    band = wid // NSTRIPE
    stripe = wid % NSTRIPE
    row_base = H_TC + band * RB
    col0 = pl.multiple_of(stripe * SW, SW)

    bufs = ((xbuf0, tbuf0, xsem0, tsem0), (xbuf1, tbuf1, xsem1, tsem1))
    nsteps = BATCH * NCHUNK

    def issue(step, bufset):
        s = jnp.minimum(step, nsteps - 1)
        b = s // NCHUNK
        j = s - b * NCHUNK
        row0 = pl.multiple_of(row_base + j * RPC, 8)
        cx = pltpu.async_copy(
            x_hbm.at[b, :, pl.ds(row0, RPC), pl.ds(col0, SW)],
            bufset[0], bufset[2])
        ct = pltpu.async_copy(
            t_hbm.at[b, pl.ds(row0, RPC), pl.ds(col0, SW)],
            bufset[1], bufset[3])
        return cx, ct

    issue(0, bufs[0])
    issue(1, bufs[1])

    def ring(g, carry):
        acc = carry
        for par in range(2):
            bufset = bufs[par]
            pltpu.make_async_copy(
                x_hbm.at[0, :, pl.ds(0, RPC), pl.ds(0, SW)],
                bufset[0], bufset[2]).wait()
            pltpu.make_async_copy(
                t_hbm.at[0, pl.ds(0, RPC), pl.ds(0, SW)],
                bufset[1], bufset[3]).wait()
            acc = _chunk_loop(bufset[0], bufset[1], acc)
            issue(2 * g + par + 2, bufset)
        return acc

    acc = lax.fori_loop(
        0, nsteps // 2, ring,
        (jnp.zeros((L,), jnp.float32), jnp.zeros((L,), jnp.float32)))

    for bufset in bufs:
        pltpu.make_async_copy(
            x_hbm.at[0, :, pl.ds(0, RPC), pl.ds(0, SW)],
            bufset[0], bufset[2]).wait()
        pltpu.make_async_copy(
            t_hbm.at[0, pl.ds(0, RPC), pl.ds(0, SW)],
            bufset[1], bufset[3]).wait()

    accbuf[0, pl.ds(0, L)] = acc[0]
    accbuf[1, pl.ds(0, L)] = acc[1]
    pltpu.sync_copy(accbuf, out_hbm.at[wid])


def _sc_loss(x, t):
    mesh = plsc.VectorSubcoreMesh(core_axis_name="c", subcore_axis_name="s")
    run = functools.partial(
        pl.kernel,
        out_type=jax.ShapeDtypeStruct((NW, 2, L), jnp.float32),
        mesh=mesh,
        compiler_params=pltpu.CompilerParams(needs_layout_passes=False, skip_device_barrier=True),
        scratch_types=[
            pltpu.VMEM((NUM_CLASSES, RPC, SW), jnp.float32),
            pltpu.VMEM((NUM_CLASSES, RPC, SW), jnp.float32),
            pltpu.VMEM((RPC, SW), jnp.int32),
            pltpu.VMEM((RPC, SW), jnp.int32),
            pltpu.VMEM((2, L), jnp.float32),
            pltpu.SemaphoreType.DMA,
            pltpu.SemaphoreType.DMA,
            pltpu.SemaphoreType.DMA,
            pltpu.SemaphoreType.DMA,
        ],
    )(_sc_body)
    return run(x, t)


def _tc_body(x_ref, t_ref, out_ref):
    b = pl.program_id(0)
    h = pl.program_id(1)

    acc_l = jnp.zeros((8, W), jnp.float32)
    acc_c = jnp.zeros((8, W), jnp.float32)
    for h8 in range(HB // 8):
        r0 = h8 * 8
        t8 = t_ref[0, pl.ds(r0, 8), :]
        m = x_ref[0, 0, pl.ds(r0, 8), :]
        for c in range(1, NUM_CLASSES):
            m = jnp.maximum(m, x_ref[0, c, pl.ds(r0, 8), :])
        z = jnp.zeros((8, W), jnp.float32)
        s = jnp.zeros((8, W), jnp.float32)
        for c in range(NUM_CLASSES):
            xc = x_ref[0, c, pl.ds(r0, 8), :]
            z = z + jnp.exp(xc - m)
            s = s + jnp.where(t8 == c, xc, 0.0)
        p = jnp.exp(s - m) / z
        p = jnp.clip(p, EPS, 1.0 - EPS)
        log_p = jnp.log(p)
        valid = t8 != IGNORE
        acc_l = acc_l + jnp.where(valid, -(1.0 - p) * log_p, 0.0)
        acc_c = acc_c + jnp.where(valid, 1.0, 0.0)

    @pl.when(jnp.logical_and(b == 0, h == 0))
    def _():
        out_ref[...] = jnp.zeros_like(out_ref)

    out_ref[0] += acc_l
    out_ref[1] += acc_c


def _tc_loss(x, t):
    return pl.pallas_call(
        _tc_body,
        grid=(BATCH, TC_GRID_H),
        in_specs=[
            pl.BlockSpec((1, NUM_CLASSES, HB, W), lambda b, h: (b, 0, h, 0)),
            pl.BlockSpec((1, HB, W), lambda b, h: (b, h, 0)),
        ],
        out_specs=pl.BlockSpec((2, 8, W), lambda b, h: (0, 0, 0)),
        out_shape=jax.ShapeDtypeStruct((2, 8, W), jnp.float32),
    )(x, t)


@jax.jit
def _loss(input, target):
    sc_parts = _sc_loss(input, target)
    tc_parts = _tc_loss(input, target)
    total = jnp.sum(sc_parts[:, 0, :]) + jnp.sum(tc_parts[0])
    count = jnp.sum(sc_parts[:, 1, :]) + jnp.sum(tc_parts[1])
    return total / jnp.maximum(count, 1.0)


def kernel(input, target):
    return _loss(input, target)


# speedup vs baseline: 1.2092x; 1.0288x over previous
import functools

import jax
import jax.numpy as jnp
from jax import lax
from jax.experimental import pallas as pl
from jax.experimental.pallas import tpu as pltpu
from jax.experimental.pallas import tpu_sc as plsc

NUM_CLASSES = 19
GAMMA = 1.0
EPS = 1e-07
IGNORE = 255

LN2 = 0.6931471805599453
SQRT2 = 1.4142135623730951

NC = 2
NS = 16
NW = NC * NS
L = 16

BATCH = 4
H = 512
W = 512

H_SC = 128
H_TC = H - H_SC

NBAND = 8
NSTRIPE = 4
RB = H_SC // NBAND
SW = W // NSTRIPE
RPC = 8
NCHUNK = RB // RPC
CHUNK_VECS = RPC * (SW // L)

HB = 192
TC_GRID_H = H_TC // HB


def _log_f32(p):
    bits = lax.bitcast_convert_type(p, jnp.int32)
    e = ((bits >> 23) & 0xFF) - 127
    mbits = (bits & 0x7FFFFF) | (127 << 23)
    m = lax.bitcast_convert_type(mbits, jnp.float32)
    big = m > SQRT2
    m = jnp.where(big, m * 0.5, m)
    e = e + jnp.where(big, jnp.ones_like(e), jnp.zeros_like(e))
    ef = e.astype(jnp.float32)
    u = (m - 1.0) / (m + 1.0)
    u2 = u * u
    poly = 2.0 * u * (1.0 + u2 * (1.0 / 3.0 + u2 * (1.0 / 5.0 + u2 * (1.0 / 7.0))))
    return ef * LN2 + poly


def _pixel_vec(xbuf, tbuf, r, col, lane_iota):
    xs = [xbuf[cls, r, pl.ds(col, L)] for cls in range(NUM_CLASSES)]
    t = tbuf[r, pl.ds(col, L)]
    m = xs[0]
    for cls in range(1, NUM_CLASSES):
        m = jnp.maximum(m, xs[cls])
    z = jnp.exp(xs[0] - m)
    for cls in range(1, NUM_CLASSES):
        z = z + jnp.exp(xs[cls] - m)
    tg = jnp.minimum(t, NUM_CLASSES - 1)
    r_vec = jnp.zeros((L,), jnp.int32) + r
    s = plsc.load_gather(xbuf, [tg, r_vec, col + lane_iota])
    p = jnp.exp(s - m) / z
    p = jnp.minimum(jnp.maximum(p, EPS), 1.0 - EPS)
    log_p = _log_f32(p)
    valid = t != IGNORE
    zero = jnp.zeros((L,), jnp.float32)
    one = jnp.ones((L,), jnp.float32)
    contrib = jnp.where(valid, (1.0 - p) * log_p, zero)
    return contrib, jnp.where(valid, one, zero)


def _chunk_loop(xbuf, tbuf, carry):
    lane_iota = lax.iota(jnp.int32, L)
    cpr = SW // L

    def it(i, c):
        al, ac = c
        for u in range(2):
            v = i * 2 + u
            r = v // cpr
            col = (v % cpr) * L
            contrib, cnt = _pixel_vec(xbuf, tbuf, r, col, lane_iota)
            al = al - contrib
            ac = ac + cnt
        return al, ac

    return lax.fori_loop(0, CHUNK_VECS // 2, it, carry)


def _sc_body(x_hbm, t_hbm, out_hbm, xbuf0, xbuf1, tbuf0, tbuf1, accbuf,
             xsem0, xsem1, tsem0, tsem1):
    cid = lax.axis_index("c")
    sid = lax.axis_index("s")
    wid = sid * NC + cid
    band = wid // NSTRIPE
    stripe = wid % NSTRIPE
    row_base = H_TC + band * RB
    col0 = pl.multiple_of(stripe * SW, SW)

    bufs = ((xbuf0, tbuf0, xsem0, tsem0), (xbuf1, tbuf1, xsem1, tsem1))
    nsteps = BATCH * NCHUNK

    def issue(step, bufset):
        s = jnp.minimum(step, nsteps - 1)
        b = s // NCHUNK
        j = s - b * NCHUNK
        row0 = pl.multiple_of(row_base + j * RPC, 8)
        cx = pltpu.async_copy(
            x_hbm.at[b, :, pl.ds(row0, RPC), pl.ds(col0, SW)],
            bufset[0], bufset[2])
        ct = pltpu.async_copy(
            t_hbm.at[b, pl.ds(row0, RPC), pl.ds(col0, SW)],
            bufset[1], bufset[3])
        return cx, ct

    issue(0, bufs[0])
    issue(1, bufs[1])

    def ring(g, carry):
        acc = carry
        for par in range(2):
            bufset = bufs[par]
            pltpu.make_async_copy(
                x_hbm.at[0, :, pl.ds(0, RPC), pl.ds(0, SW)],
                bufset[0], bufset[2]).wait()
            pltpu.make_async_copy(
                t_hbm.at[0, pl.ds(0, RPC), pl.ds(0, SW)],
                bufset[1], bufset[3]).wait()
            acc = _chunk_loop(bufset[0], bufset[1], acc)
            issue(2 * g + par + 2, bufset)
        return acc

    acc = lax.fori_loop(
        0, nsteps // 2, ring,
        (jnp.zeros((L,), jnp.float32), jnp.zeros((L,), jnp.float32)))

    for bufset in bufs:
        pltpu.make_async_copy(
            x_hbm.at[0, :, pl.ds(0, RPC), pl.ds(0, SW)],
            bufset[0], bufset[2]).wait()
        pltpu.make_async_copy(
            t_hbm.at[0, pl.ds(0, RPC), pl.ds(0, SW)],
            bufset[1], bufset[3]).wait()

    accbuf[0, pl.ds(0, L)] = acc[0]
    accbuf[1, pl.ds(0, L)] = acc[1]
    pltpu.sync_copy(accbuf, out_hbm.at[wid])


def _sc_loss(x, t):
    mesh = plsc.VectorSubcoreMesh(core_axis_name="c", subcore_axis_name="s")
    run = functools.partial(
        pl.kernel,
        out_type=jax.ShapeDtypeStruct((NW, 2, L), jnp.float32),
        mesh=mesh,
        compiler_params=pltpu.CompilerParams(needs_layout_passes=False, skip_device_barrier=True),
        scratch_types=[
            pltpu.VMEM((NUM_CLASSES, RPC, SW), jnp.float32),
            pltpu.VMEM((NUM_CLASSES, RPC, SW), jnp.float32),
            pltpu.VMEM((RPC, SW), jnp.int32),
            pltpu.VMEM((RPC, SW), jnp.int32),
            pltpu.VMEM((2, L), jnp.float32),
            pltpu.SemaphoreType.DMA,
            pltpu.SemaphoreType.DMA,
            pltpu.SemaphoreType.DMA,
            pltpu.SemaphoreType.DMA,
        ],
    )(_sc_body)
    return run(x, t)


def _tc_body(x_ref, t_ref, out_ref):
    b = pl.program_id(0)
    h = pl.program_id(1)

    acc_l = jnp.zeros((8, W), jnp.float32)
    acc_c = jnp.zeros((8, W), jnp.float32)
    for h8 in range(HB // 8):
        r0 = h8 * 8
        t8 = t_ref[0, pl.ds(r0, 8), :]
        m = x_ref[0, 0, pl.ds(r0, 8), :]
        for c in range(1, NUM_CLASSES):
            m = jnp.maximum(m, x_ref[0, c, pl.ds(r0, 8), :])
        z = jnp.zeros((8, W), jnp.float32)
        s = jnp.zeros((8, W), jnp.float32)
        for c in range(NUM_CLASSES):
            xc = x_ref[0, c, pl.ds(r0, 8), :]
            z = z + jnp.exp(xc - m)
            s = s + jnp.where(t8 == c, xc, 0.0)
        p = jnp.exp(s - m) / z
        p = jnp.clip(p, EPS, 1.0 - EPS)
        log_p = jnp.log(p)
        valid = t8 != IGNORE
        acc_l = acc_l + jnp.where(valid, -(1.0 - p) * log_p, 0.0)
        acc_c = acc_c + jnp.where(valid, 1.0, 0.0)

    @pl.when(jnp.logical_and(b == 0, h == 0))
    def _():
        out_ref[...] = jnp.zeros_like(out_ref)

    out_ref[0] += acc_l
    out_ref[1] += acc_c


def _tc_loss(x, t):
    return pl.pallas_call(
        _tc_body,
        grid=(BATCH, TC_GRID_H),
        in_specs=[
            pl.BlockSpec((1, NUM_CLASSES, HB, W), lambda b, h: (b, 0, h, 0)),
            pl.BlockSpec((1, HB, W), lambda b, h: (b, h, 0)),
        ],
        out_specs=pl.BlockSpec((2, 8, W), lambda b, h: (0, 0, 0)),
        out_shape=jax.ShapeDtypeStruct((2, 8, W), jnp.float32),
    )(x, t)


@jax.jit
def _loss(input, target):
    sc_parts = _sc_loss(input, target)
    tc_parts = _tc_loss(input, target)
    total = jnp.sum(sc_parts[:, 0, :]) + jnp.sum(tc_parts[0])
    count = jnp.sum(sc_parts[:, 1, :]) + jnp.sum(tc_parts[1])
    return total / jnp.maximum(count, 1.0)


def kernel(input, target):
    return _loss(input, target)
